# baseline (speedup 1.0000x reference)
import jax
import jax.numpy as jnp
import numpy as np
from jax.experimental import pallas as pl
from jax.experimental.pallas import tpu as pltpu

_FACTORS = (4, 4, 16, 5, 3, 5, 5, 6, 7, 4)
_ADD = tuple(np.concatenate([[0], np.cumsum(_FACTORS)[:-1]]).tolist())
_NCH = sum(_FACTORS)
_BM = 16


def _decode_kernel(codes_ref, out_ref):
    q = codes_ref[...].astype(jnp.float32)
    lo = jnp.zeros(q.shape, jnp.int32)
    hi = jnp.zeros(q.shape, jnp.int32)
    for c, f in enumerate(_FACTORS):
        if f & (f - 1) == 0:
            qn = jnp.floor(q * (1.0 / f))
        else:
            qn = jnp.floor((q + 0.5) * (1.0 / f))
        d = q - f * qn
        q = qn
        pos = d.astype(jnp.int32)
        if _ADD[c] + f <= 32:
            lo = lo | (1 << (pos + _ADD[c]))
        else:
            hi = hi | (1 << (pos + (_ADD[c] - 32)))
    for j in range(_NCH):
        w = lo if j < 32 else hi
        sh = j if j < 32 else j - 32
        out_ref[:, j, :, :] = ((w >> sh) & 1).astype(jnp.float32)


@jax.jit
def kernel(codes):
    batch = codes.shape[0]
    codes4 = codes.reshape(batch, 11, 15)
    grid = (batch // _BM,)
    return pl.pallas_call(
        _decode_kernel,
        grid=grid,
        in_specs=[pl.BlockSpec((_BM, 11, 15), lambda i: (i, 0, 0))],
        out_specs=pl.BlockSpec((_BM, _NCH, 11, 15), lambda i: (i, 0, 0, 0)),
        out_shape=jax.ShapeDtypeStruct((batch, _NCH, 11, 15), jnp.float32),
        compiler_params=pltpu.CompilerParams(
            dimension_semantics=("parallel",),
        ),
    )(codes4)


# speedup vs baseline: 4.8398x; 4.8398x over previous
import jax
import jax.numpy as jnp
import numpy as np
from jax.experimental import pallas as pl
from jax.experimental.pallas import tpu as pltpu

_FACTORS = (4, 4, 16, 5, 3, 5, 5, 6, 7, 4)
_ADD = tuple(np.concatenate([[0], np.cumsum(_FACTORS)[:-1]]).tolist())
_NCH = sum(_FACTORS)
_K = 165
_BM = 64
_NOUT = _NCH * _K


def _decode_kernel(codes_ref, shamt_ref, out_ref):
    q = codes_ref[...].astype(jnp.float32)
    lo = jnp.zeros(q.shape, jnp.int32)
    hi = jnp.zeros(q.shape, jnp.int32)
    for c, f in enumerate(_FACTORS):
        if f & (f - 1) == 0:
            qn = jnp.floor(q * (1.0 / f))
        else:
            qn = jnp.floor((q + 0.5) * (1.0 / f))
        d = q - f * qn
        q = qn
        pos = d.astype(jnp.int32)
        if _ADD[c] + f <= 32:
            lo = lo | (1 << (pos + _ADD[c]))
        else:
            hi = hi | (1 << (pos + (_ADD[c] - 32)))
    m = jnp.concatenate([lo] * 32 + [hi] * (_NCH - 32), axis=1)
    out_ref[...] = ((m >> shamt_ref[...]) & 1).astype(jnp.float32)


@jax.jit
def kernel(codes):
    batch = codes.shape[0]
    grid = (batch // _BM,)
    shamt = (np.arange(_NOUT, dtype=np.int32) // _K) & 31
    shamt = jnp.asarray(shamt)[None, :]
    out = pl.pallas_call(
        _decode_kernel,
        grid=grid,
        in_specs=[
            pl.BlockSpec((_BM, _K), lambda i: (i, 0)),
            pl.BlockSpec((1, _NOUT), lambda i: (0, 0)),
        ],
        out_specs=pl.BlockSpec((_BM, _NOUT), lambda i: (i, 0)),
        out_shape=jax.ShapeDtypeStruct((batch, _NOUT), jnp.float32),
        compiler_params=pltpu.CompilerParams(
            dimension_semantics=("parallel",),
        ),
    )(codes, shamt)
    return out.reshape(batch, _NCH, 11, 15)


# speedup vs baseline: 30.5471x; 6.3117x over previous
import jax
import jax.numpy as jnp
import numpy as np
from jax.experimental import pallas as pl
from jax.experimental.pallas import tpu as pltpu

_FACTORS = (4, 4, 16, 5, 3, 5, 5, 6, 7, 4)
_ADD = tuple(np.concatenate([[0], np.cumsum(_FACTORS)[:-1]]).tolist())
_NCH = sum(_FACTORS)
_K = 165
_BL = 256


def _decode_kernel(codes_ref, out_ref):
    q = codes_ref[...].astype(jnp.float32)
    lo = jnp.zeros(q.shape, jnp.int32)
    hi = jnp.zeros(q.shape, jnp.int32)
    for c, f in enumerate(_FACTORS):
        if f & (f - 1) == 0:
            qn = jnp.floor(q * (1.0 / f))
        else:
            qn = jnp.floor((q + 0.5) * (1.0 / f))
        d = q - f * qn
        q = qn
        pos = d.astype(jnp.int32)
        if _ADD[c] + f <= 32:
            lo = lo | (1 << (pos + _ADD[c]))
        else:
            hi = hi | (1 << (pos + (_ADD[c] - 32)))
    for r in range(11):
        lo_r = lo[15 * r:15 * r + 15, :]
        hi_r = hi[15 * r:15 * r + 15, :]
        for j in range(_NCH):
            w = lo_r if j < 32 else hi_r
            sh = j if j < 32 else j - 32
            out_ref[j, r, :, :] = ((w >> sh) & 1).astype(jnp.float32)


@jax.jit
def kernel(codes):
    batch = codes.shape[0]
    ct = codes.T
    grid = (batch // _BL,)
    out_t = pl.pallas_call(
        _decode_kernel,
        grid=grid,
        in_specs=[pl.BlockSpec((_K, _BL), lambda i: (0, i))],
        out_specs=pl.BlockSpec((_NCH, 11, 15, _BL), lambda i: (0, 0, 0, i)),
        out_shape=jax.ShapeDtypeStruct((_NCH, 11, 15, batch), jnp.float32),
        compiler_params=pltpu.CompilerParams(
            dimension_semantics=("parallel",),
        ),
    )(ct)
    return out_t.transpose(3, 0, 1, 2)
